# baseline (device time: 49480 ns/iter reference)
import jax
import jax.numpy as jnp
from jax import lax
from jax.experimental import pallas as pl
from jax.experimental.pallas import tpu as pltpu

N_DEV = 4
B = 8
H = 8
D = 128
BS = 16
NPAGES = 512
CHUNK = 64
NCHUNK = NPAGES // CHUNK
NSLOT = 6
CKEYS = CHUNK * BS
PACK = 256


def kernel(Q, K, V, bt, lens):
    lens2 = lens.reshape(B, 1)

    def body(q_ref, k_hbm, v_hbm, bt_ref, lens_ref, out_ref,
             k_vm, v_vm, o_acc, m_acc, l_acc, gather_ref,
             k_sems, v_sems, send_sems, recv_sems):
        my_pos = lax.axis_index("i")

        def issue(j, slot):
            sl = pl.ds(j * CHUNK, CHUNK)
            pltpu.make_async_copy(
                k_hbm.at[sl], k_vm.at[slot], k_sems.at[slot]).start()
            pltpu.make_async_copy(
                v_hbm.at[sl], v_vm.at[slot], v_sems.at[slot]).start()

        for j in range(NSLOT):
            issue(j, j)

        o_acc[...] = jnp.zeros((H, B, D), jnp.float32)
        m_acc[...] = jnp.full((H, B, 1), -1e30, jnp.float32)
        l_acc[...] = jnp.zeros((H, B, 1), jnp.float32)

        bt_v = bt_ref[...]
        lens_v = lens_ref[...]
        j_idx = lax.broadcasted_iota(jnp.int32, (B, NPAGES), 1)
        valid = (j_idx < lens_v).astype(jnp.float32)
        qf = q_ref[...].reshape(B, H * D)
        scale = D ** -0.5

        def chunk_step(i, slot):
            pltpu.make_async_copy(
                k_hbm.at[pl.ds(0, CHUNK)], k_vm.at[slot], k_sems.at[slot]).wait()
            pltpu.make_async_copy(
                v_hbm.at[pl.ds(0, CHUNK)], v_vm.at[slot], v_sems.at[slot]).wait()

            base = my_pos * NPAGES + i * CHUNK
            local_id = bt_v - base
            p_iota = lax.broadcasted_iota(jnp.int32, (B, NPAGES, CHUNK), 2)
            onehot = (local_id[:, :, None] == p_iota).astype(jnp.float32)
            counts = jnp.sum(valid[:, :, None] * onehot, axis=1)
            counts_key = jnp.broadcast_to(
                counts[:, :, None], (B, CHUNK, BS)
            ).reshape(B, CKEYS)

            kf = k_vm[slot].reshape(CKEYS, H * D)
            vf = v_vm[slot].reshape(CKEYS, H * D)

            for h in range(H):
                q_h = qf[:, h * D:(h + 1) * D].astype(jnp.bfloat16)
                k_h = kf[:, h * D:(h + 1) * D].astype(jnp.bfloat16)
                v_h = vf[:, h * D:(h + 1) * D].astype(jnp.bfloat16)

                s_h = lax.dot_general(
                    q_h, k_h, (((1,), (1,)), ((), ())),
                    preferred_element_type=jnp.float32,
                ) * scale
                s_m = jnp.where(counts_key > 0, s_h, -1e30)
                m_c = jnp.max(s_m, axis=-1, keepdims=True)
                w = counts_key * jnp.exp(s_m - m_c)
                l_c = jnp.sum(w, axis=-1, keepdims=True)
                o_c = lax.dot_general(
                    w.astype(jnp.bfloat16), v_h, (((1,), (0,)), ((), ())),
                    preferred_element_type=jnp.float32,
                )

                m_old = m_acc[h]
                m_new = jnp.maximum(m_old, m_c)
                sc_old = jnp.exp(m_old - m_new)
                sc_c = jnp.exp(m_c - m_new)
                m_acc[h] = m_new
                l_acc[h] = l_acc[h] * sc_old + l_c * sc_c
                o_acc[h] = o_acc[h] * sc_old + o_c * sc_c

            if i + NSLOT < NCHUNK:
                issue(i + NSLOT, slot)

        for i in range(NCHUNK):
            chunk_step(i, i % NSLOT)

        left = lax.rem(my_pos - 1 + N_DEV, N_DEV)
        right = lax.rem(my_pos + 1, N_DEV)

        packed = jnp.concatenate(
            [o_acc[...], m_acc[...], l_acc[...],
             jnp.zeros((H, B, PACK - D - 2), jnp.float32)],
            axis=-1,
        )
        gather_ref[my_pos] = packed

        barrier_sem = pltpu.get_barrier_semaphore()
        for nbr in (left, right):
            pl.semaphore_signal(
                barrier_sem, inc=1,
                device_id=(nbr,), device_id_type=pl.DeviceIdType.MESH,
            )
        pl.semaphore_wait(barrier_sem, 2)

        for h in range(N_DEV - 1):
            slot = lax.rem(my_pos - h + N_DEV, N_DEV)
            rdma = pltpu.make_async_remote_copy(
                src_ref=gather_ref.at[slot],
                dst_ref=gather_ref.at[slot],
                send_sem=send_sems.at[h],
                recv_sem=recv_sems.at[h],
                device_id=(right,),
                device_id_type=pl.DeviceIdType.MESH,
            )
            rdma.start()
            rdma.wait()

        g = gather_ref[...]
        o_all = g[:, :, :, :D]
        m_all = g[:, :, :, D]
        l_all = g[:, :, :, D + 1]
        m_g = jnp.max(m_all, axis=0)
        coef = jnp.exp(m_all - m_g[None])
        l_tot = jnp.sum(coef * l_all, axis=0)
        o_tot = jnp.sum(coef[:, :, :, None] * o_all, axis=0)
        res = o_tot / l_tot[:, :, None]
        out_ref[...] = res.transpose(1, 0, 2)[:, None, :, :]

    return pl.pallas_call(
        body,
        out_shape=jax.ShapeDtypeStruct((B, 1, H, D), jnp.float32),
        in_specs=[
            pl.BlockSpec(memory_space=pltpu.VMEM),
            pl.BlockSpec(memory_space=pl.ANY),
            pl.BlockSpec(memory_space=pl.ANY),
            pl.BlockSpec(memory_space=pltpu.VMEM),
            pl.BlockSpec(memory_space=pltpu.VMEM),
        ],
        out_specs=pl.BlockSpec(memory_space=pltpu.VMEM),
        scratch_shapes=[
            pltpu.VMEM((NSLOT, CHUNK, BS, H, D), jnp.float32),
            pltpu.VMEM((NSLOT, CHUNK, BS, H, D), jnp.float32),
            pltpu.VMEM((H, B, D), jnp.float32),
            pltpu.VMEM((H, B, 1), jnp.float32),
            pltpu.VMEM((H, B, 1), jnp.float32),
            pltpu.VMEM((N_DEV, H, B, PACK), jnp.float32),
            pltpu.SemaphoreType.DMA((NSLOT,)),
            pltpu.SemaphoreType.DMA((NSLOT,)),
            pltpu.SemaphoreType.DMA((N_DEV - 1,)),
            pltpu.SemaphoreType.DMA((N_DEV - 1,)),
        ],
        compiler_params=pltpu.CompilerParams(
            collective_id=0, vmem_limit_bytes=60 * 1024 * 1024
        ),
    )(Q, K, V, bt, lens2)


# device time: 36867 ns/iter; 1.3421x vs baseline; 1.3421x over previous
import jax
import jax.numpy as jnp
from jax import lax
from jax.experimental import pallas as pl
from jax.experimental.pallas import tpu as pltpu

N_DEV = 4
B = 8
H = 8
D = 128
BS = 16
NPAGES = 512
CHUNK = 64
NCHUNK = NPAGES // CHUNK
NSLOT = 6
CKEYS = CHUNK * BS
HB = H * B
PACK = 256


def kernel(Q, K, V, bt, lens):
    lens2 = lens.reshape(B, 1)

    def body(q_ref, k_hbm, v_hbm, bt_ref, lens_ref, out_ref,
             k_vm, v_vm, o_acc, m_acc, l_acc, gather_ref,
             k_sems, v_sems, send_sems, recv_sems):
        my_pos = lax.axis_index("i")

        def issue(j, slot):
            sl = pl.ds(j * CHUNK, CHUNK)
            pltpu.make_async_copy(
                k_hbm.at[sl], k_vm.at[slot], k_sems.at[slot]).start()
            pltpu.make_async_copy(
                v_hbm.at[sl], v_vm.at[slot], v_sems.at[slot]).start()

        for j in range(NSLOT):
            issue(j, j)

        barrier_sem = pltpu.get_barrier_semaphore()
        for off in (1, 2, 3):
            pl.semaphore_signal(
                barrier_sem, inc=1,
                device_id=(lax.rem(my_pos + off, N_DEV),),
                device_id_type=pl.DeviceIdType.MESH,
            )

        o_acc[...] = jnp.zeros((HB, D), jnp.float32)
        m_acc[...] = jnp.full((1, HB), -1e30, jnp.float32)
        l_acc[...] = jnp.zeros((1, HB), jnp.float32)

        bt_v = bt_ref[...]
        lens_v = lens_ref[...]
        scale = D ** -0.5

        base0 = my_pos * NPAGES
        p_col = lax.broadcasted_iota(jnp.int32, (NPAGES, NPAGES), 0) + base0
        j_row = lax.broadcasted_iota(jnp.int32, (1, NPAGES), 1)
        cnt_cols = []
        for b in range(B):
            vmask = (j_row < lens_v[b:b + 1, 0:1]).astype(jnp.float32)
            eq = (bt_v[b:b + 1, :] == p_col).astype(jnp.float32)
            cnt_cols.append(jnp.sum(eq * vmask, axis=1, keepdims=True))
        counts_all = jnp.concatenate(cnt_cols, axis=1)

        qf = q_ref[...].reshape(B, H * D)
        qT3 = qf.transpose(1, 0).reshape(H, D, B)
        hr = lax.broadcasted_iota(jnp.int32, (H, 1, H, 1), 0)
        hc = lax.broadcasted_iota(jnp.int32, (H, 1, H, 1), 2)
        eye = (hr == hc).astype(jnp.float32)
        Wbd = (qT3[:, :, None, :] * eye).reshape(H * D, HB)
        Wbd = (Wbd * scale).astype(jnp.bfloat16)
        eye_o = eye

        def chunk_step(i, slot):
            pltpu.make_async_copy(
                k_hbm.at[pl.ds(0, CHUNK)], k_vm.at[slot], k_sems.at[slot]).wait()
            pltpu.make_async_copy(
                v_hbm.at[pl.ds(0, CHUNK)], v_vm.at[slot], v_sems.at[slot]).wait()

            counts_t = counts_all[i * CHUNK:(i + 1) * CHUNK, :]
            ck = jnp.broadcast_to(
                counts_t[:, None, :], (CHUNK, BS, B)
            ).reshape(CKEYS, B)
            ck = jnp.broadcast_to(
                ck[:, None, :], (CKEYS, H, B)
            ).reshape(CKEYS, HB)

            kc = k_vm[slot].reshape(CKEYS, H * D).astype(jnp.bfloat16)
            vc = v_vm[slot].reshape(CKEYS, H * D).astype(jnp.bfloat16)

            s_t = lax.dot_general(
                kc, Wbd, (((1,), (0,)), ((), ())),
                preferred_element_type=jnp.float32,
            )
            s_m = jnp.where(ck > 0, s_t, -1e30)
            m_c = jnp.max(s_m, axis=0, keepdims=True)
            w_t = ck * jnp.exp(s_m - m_c)
            l_c = jnp.sum(w_t, axis=0, keepdims=True)

            w_T = w_t.transpose(1, 0).astype(jnp.bfloat16)
            o_full = lax.dot_general(
                w_T, vc, (((1,), (0,)), ((), ())),
                preferred_element_type=jnp.float32,
            )
            F = o_full.reshape(H, B, H, D)
            o_c = jnp.sum(F * eye_o, axis=2).reshape(HB, D)

            m_old = m_acc[...]
            m_new = jnp.maximum(m_old, m_c)
            sc_old = jnp.exp(m_old - m_new)
            sc_c = jnp.exp(m_c - m_new)
            m_acc[...] = m_new
            l_acc[...] = l_acc[...] * sc_old + l_c * sc_c
            o_acc[...] = (o_acc[...] * sc_old.transpose(1, 0)
                          + o_c * sc_c.transpose(1, 0))

            if i + NSLOT < NCHUNK:
                issue(i + NSLOT, slot)

        for i in range(NCHUNK):
            chunk_step(i, i % NSLOT)

        packed = jnp.concatenate(
            [o_acc[...],
             m_acc[...].transpose(1, 0),
             l_acc[...].transpose(1, 0),
             jnp.zeros((HB, PACK - D - 2), jnp.float32)],
            axis=-1,
        )
        gather_ref[my_pos] = packed

        pl.semaphore_wait(barrier_sem, 3)

        sends = []
        for j, off in enumerate((1, 2, 3)):
            rdma = pltpu.make_async_remote_copy(
                src_ref=gather_ref.at[my_pos],
                dst_ref=gather_ref.at[my_pos],
                send_sem=send_sems.at[j],
                recv_sem=recv_sems.at[my_pos],
                device_id=(lax.rem(my_pos + off, N_DEV),),
                device_id_type=pl.DeviceIdType.MESH,
            )
            rdma.start()
            sends.append(rdma)

        for off in (1, 2, 3):
            src = lax.rem(my_pos + off, N_DEV)
            pltpu.make_async_remote_copy(
                src_ref=gather_ref.at[src],
                dst_ref=gather_ref.at[src],
                send_sem=send_sems.at[0],
                recv_sem=recv_sems.at[src],
                device_id=(src,),
                device_id_type=pl.DeviceIdType.MESH,
            ).wait_recv()
        for rdma in sends:
            rdma.wait_send()

        g = gather_ref[...]
        o_all = g[:, :, :D]
        m_all = g[:, :, D]
        l_all = g[:, :, D + 1]
        m_g = jnp.max(m_all, axis=0)
        coef = jnp.exp(m_all - m_g[None])
        l_tot = jnp.sum(coef * l_all, axis=0)
        o_tot = jnp.sum(coef[:, :, None] * o_all, axis=0)
        res = (o_tot / l_tot[:, None]).reshape(H, B, D)
        out_ref[...] = res.transpose(1, 0, 2)[:, None, :, :]

    return pl.pallas_call(
        body,
        out_shape=jax.ShapeDtypeStruct((B, 1, H, D), jnp.float32),
        in_specs=[
            pl.BlockSpec(memory_space=pltpu.VMEM),
            pl.BlockSpec(memory_space=pl.ANY),
            pl.BlockSpec(memory_space=pl.ANY),
            pl.BlockSpec(memory_space=pltpu.VMEM),
            pl.BlockSpec(memory_space=pltpu.VMEM),
        ],
        out_specs=pl.BlockSpec(memory_space=pltpu.VMEM),
        scratch_shapes=[
            pltpu.VMEM((NSLOT, CHUNK, BS, H, D), jnp.float32),
            pltpu.VMEM((NSLOT, CHUNK, BS, H, D), jnp.float32),
            pltpu.VMEM((HB, D), jnp.float32),
            pltpu.VMEM((1, HB), jnp.float32),
            pltpu.VMEM((1, HB), jnp.float32),
            pltpu.VMEM((N_DEV, HB, PACK), jnp.float32),
            pltpu.SemaphoreType.DMA((NSLOT,)),
            pltpu.SemaphoreType.DMA((NSLOT,)),
            pltpu.SemaphoreType.DMA((N_DEV - 1,)),
            pltpu.SemaphoreType.DMA((N_DEV,)),
        ],
        compiler_params=pltpu.CompilerParams(
            collective_id=0, vmem_limit_bytes=60 * 1024 * 1024
        ),
    )(Q, K, V, bt, lens2)
